# Initial kernel scaffold; baseline (speedup 1.0000x reference)
import jax
import jax.numpy as jnp
from jax.experimental import pallas as pl


def kernel(x, edge_index, edge_attr, distances, distance_indices, phis, phi_indices, psis, psi_indices, node_map, LS_map, alpha_indices, params):
    raise NotImplementedError("write your pallas kernel here")


# speedup vs baseline: 1.6284x; 1.6284x over previous
import functools

import jax
import jax.numpy as jnp
from jax import lax
from jax.experimental import pallas as pl
from jax.experimental.pallas import tpu as pltpu

N_NODES = 10000
N_MOL = 512
N_LS = 4096
HEADS = 4
F_H = 64

BE = 2048


def _pad_rows(a, n):
    return jnp.pad(a, ((0, n - a.shape[0]),) + ((0, 0),) * (a.ndim - 1))


def _pad_idx(idx, n, val):
    return jnp.pad(idx, (0, n - idx.shape[0]), constant_values=val)


def _leaky(x, s):
    return jnp.maximum(x, s * x)


def _gather_rows(table, idx):
    return jnp.take(table, idx, axis=0)


def _scatter_add(rows, idx, s_pad):
    full = jax.ops.segment_sum(rows, idx, num_segments=s_pad)
    return jnp.stack([full, jnp.zeros_like(full)])


def _t1_body(ea, xs, w0, b0, w1, b1, msg_out):
    eh = jnp.maximum(_dot3(ea[...], w0[...]) + b0[...], 0.0)
    ewf = _dot3(eh, w1[...]) + b1[...]
    ewf = ewf.astype(jnp.bfloat16).astype(jnp.float32)
    xsv = xs[...].astype(jnp.bfloat16).astype(jnp.float32)
    acc = xsv[:, 0:1] * ewf[:, 0:32]
    for i in range(1, 16):
        acc = acc + xsv[:, i:i + 1] * ewf[:, i * 32:(i + 1) * 32]
    msg_out[...] = acc


def _t1_msg(ea, xs, p):
    ep = ea.shape[0]
    grid = ep // BE
    w0, b0 = p['econv_mlp'][0]
    w1, b1 = p['econv_mlp'][1]
    return pl.pallas_call(
        _t1_body,
        grid=(grid,),
        in_specs=[
            pl.BlockSpec((BE, 16), lambda i: (i, 0)),
            pl.BlockSpec((BE, 16), lambda i: (i, 0)),
            pl.BlockSpec((16, 32), lambda i: (0, 0)),
            pl.BlockSpec((1, 32), lambda i: (0, 0)),
            pl.BlockSpec((32, 512), lambda i: (0, 0)),
            pl.BlockSpec((1, 512), lambda i: (0, 0)),
        ],
        out_specs=pl.BlockSpec((BE, 32), lambda i: (i, 0)),
        out_shape=jax.ShapeDtypeStruct((ep, 32), jnp.float32),
    )(ea, xs, w0, b0[None, :], w1, b1[None, :])


BN = 2000


def _t2a_body(x, agg_a, agg_b, root, cbias, gat_w, asrc, adst, hh_out, pk_out):
    h = _dot3(x[...], root[...]) + agg_a[...] + agg_b[...] + cbias[...]
    hh = _dot3(h, gat_w[...])
    hh_out[...] = hh
    cols = []
    for hd in range(HEADS):
        blk = hh[:, hd * F_H:(hd + 1) * F_H]
        cols.append(jnp.sum(blk * asrc[hd:hd + 1, :], axis=1, keepdims=True))
    for hd in range(HEADS):
        blk = hh[:, hd * F_H:(hd + 1) * F_H]
        cols.append(jnp.sum(blk * adst[hd:hd + 1, :], axis=1, keepdims=True))
    pk_out[...] = jnp.concatenate(
        cols + [jnp.zeros((BN, 8), jnp.float32)], axis=1)


def _t2b_body(pk, out):
    a_s = pk[:, 0:4]
    a_d = pk[:, 4:8]
    amax = jnp.max(a_s, axis=0, keepdims=True)
    m_hat = 0.5 * (_leaky(a_s + a_d, 0.2) + _leaky(a_d + amax, 0.2))
    out[...] = jnp.concatenate(
        [a_s, a_d, m_hat, jnp.zeros((N_NODES, 4), jnp.float32)], axis=1)


def _t2_node(x, agg_a, agg_b, p):
    hh, pk0 = pl.pallas_call(
        _t2a_body,
        grid=(N_NODES // BN,),
        in_specs=[pl.BlockSpec((BN, 16), lambda i: (i, 0)),
                  pl.BlockSpec((BN, 32), lambda i: (i, 0)),
                  pl.BlockSpec((BN, 32), lambda i: (i, 0)),
                  pl.BlockSpec((16, 32), lambda i: (0, 0)),
                  pl.BlockSpec((1, 32), lambda i: (0, 0)),
                  pl.BlockSpec((32, HEADS * F_H), lambda i: (0, 0)),
                  pl.BlockSpec((8, 64), lambda i: (0, 0)),
                  pl.BlockSpec((8, 64), lambda i: (0, 0))],
        out_specs=(pl.BlockSpec((BN, HEADS * F_H), lambda i: (i, 0)),
                   pl.BlockSpec((BN, 16), lambda i: (i, 0))),
        out_shape=(jax.ShapeDtypeStruct((N_NODES, HEADS * F_H), jnp.float32),
                   jax.ShapeDtypeStruct((N_NODES, 16), jnp.float32)),
    )(x, agg_a[0:N_NODES], agg_b[0:N_NODES], p['econv_root'],
      p['econv_bias'][None, :], p['gat_W'],
      jnp.pad(p['gat_asrc'], ((0, 4), (0, 0))),
      jnp.pad(p['gat_adst'], ((0, 4), (0, 0))))
    packed = pl.pallas_call(
        _t2b_body,
        out_shape=jax.ShapeDtypeStruct((N_NODES, 16), jnp.float32),
    )(pk0)
    return hh, packed


def _t3_body(gs, gd, w_out):
    t = gs[:, 0:4] + gd[:, 4:8]
    lg = _leaky(t, 0.2)
    w = jnp.exp(lg - gd[:, 8:12])
    w_out[...] = jnp.concatenate([w, jnp.zeros((BE, 12), jnp.float32)], axis=1)


def _t3_gatw(gs, gd):
    ep = gs.shape[0]
    return pl.pallas_call(
        _t3_body,
        grid=(ep // BE,),
        in_specs=[pl.BlockSpec((BE, 16), lambda i: (i, 0)),
                  pl.BlockSpec((BE, 16), lambda i: (i, 0))],
        out_specs=pl.BlockSpec((BE, 16), lambda i: (i, 0)),
        out_shape=jax.ShapeDtypeStruct((ep, 16), jnp.float32),
    )(gs, gd)


def _t4_body(w, da, db, hhg, y_out):
    den = da[:, 0:4] + db[:, 0:4]
    alpha = w[:, 0:4] / den
    acc = alpha[:, 0:1] * hhg[:, 0:F_H]
    for hd in range(1, HEADS):
        acc = acc + alpha[:, hd:hd + 1] * hhg[:, hd * F_H:(hd + 1) * F_H]
    y_out[...] = acc


def _t4_y(w, da, db, hhg):
    ep = w.shape[0]
    return pl.pallas_call(
        _t4_body,
        grid=(ep // BE,),
        in_specs=[pl.BlockSpec((BE, 16), lambda i: (i, 0)),
                  pl.BlockSpec((BE, 16), lambda i: (i, 0)),
                  pl.BlockSpec((BE, 16), lambda i: (i, 0)),
                  pl.BlockSpec((BE, HEADS * F_H), lambda i: (i, 0))],
        out_specs=pl.BlockSpec((BE, F_H), lambda i: (i, 0)),
        out_shape=jax.ShapeDtypeStruct((ep, F_H), jnp.float32),
    )(w, da, db, hhg)


def _t5_body(pa, pb, gb, h_out):
    h_out[...] = (pa[0:N_NODES, :] + pb[0:N_NODES, :]) / HEADS + gb[...]


def _t5_h(pa, pb, p):
    return pl.pallas_call(
        _t5_body,
        out_shape=jax.ShapeDtypeStruct((N_NODES, F_H), jnp.float32),
    )(pa, pb, p['gat_b'][None, :])


def _dot3(a, b):
    return jnp.dot(a, b, preferred_element_type=jnp.float32)


def _mlp23(l1, w2, b2, w3, b3):
    h1 = _leaky(l1, 0.01)
    h2 = _leaky(_dot3(h1, w2) + b2, 0.01)
    return _dot3(h2, w3) + b3


def _t6d_body(hi, hj, d, wa, wb, wd, b1, w2, b2, w3, b3, z_out):
    vi, vj = hi[...], hj[...]
    s = jnp.concatenate([vi, vj], axis=0)
    t = jnp.concatenate([vj, vi], axis=0)
    ex = d[...] * wd[...] + b1[...]
    l1 = _dot3(s, wa[...]) + _dot3(t, wb[...]) + jnp.concatenate([ex, ex], 0)
    zz = _mlp23(l1, w2[...], b2[...], w3[...], b3[...])
    z = zz[0:BE] + zz[BE:2 * BE]
    z_out[...] = jnp.concatenate([z, jnp.zeros((BE, 8), jnp.float32)], axis=1)


def _t6_d(hi, hj, d, p):
    ep = hi.shape[0]
    (w1, b1), (w2, b2), (w3, b3) = p['mlp_D']
    return pl.pallas_call(
        _t6d_body,
        grid=(ep // BE,),
        in_specs=[pl.BlockSpec((BE, F_H), lambda i: (i, 0)),
                  pl.BlockSpec((BE, F_H), lambda i: (i, 0)),
                  pl.BlockSpec((BE, 1), lambda i: (i, 0)),
                  pl.BlockSpec((F_H, 64), lambda i: (0, 0)),
                  pl.BlockSpec((F_H, 64), lambda i: (0, 0)),
                  pl.BlockSpec((1, 64), lambda i: (0, 0)),
                  pl.BlockSpec((1, 64), lambda i: (0, 0)),
                  pl.BlockSpec((64, 64), lambda i: (0, 0)),
                  pl.BlockSpec((1, 64), lambda i: (0, 0)),
                  pl.BlockSpec((64, 8), lambda i: (0, 0)),
                  pl.BlockSpec((1, 8), lambda i: (0, 0))],
        out_specs=pl.BlockSpec((BE, 16), lambda i: (i, 0)),
        out_shape=jax.ShapeDtypeStruct((ep, 16), jnp.float32),
    )(hi, hj, d, w1[0:64], w1[64:128], w1[128:129], b1[None, :], w2,
      b2[None, :], w3, b3[None, :])


def _t6p_body(hi, hj, hk, trig, wa, wb, wc, wt, b1, w2, b2, w3, b3, z_out):
    vi, vj, vk = hi[...], hj[...], hk[...]
    tv = trig[...]
    ex = tv[:, 0:1] * wt[0:1, :] + tv[:, 1:2] * wt[1:2, :] + b1[...]
    s = jnp.concatenate([vi, vk], axis=0)
    t = jnp.concatenate([vj, vj], axis=0)
    u = jnp.concatenate([vk, vi], axis=0)
    l1 = (_dot3(s, wa[...]) + _dot3(t, wb[...]) + _dot3(u, wc[...])
          + jnp.concatenate([ex, ex], 0))
    zz = _mlp23(l1, w2[...], b2[...], w3[...], b3[...])
    z = zz[0:BE] + zz[BE:2 * BE]
    z_out[...] = jnp.concatenate([z, jnp.zeros((BE, 8), jnp.float32)], axis=1)


def _t6_phi(hi, hj, hk, trig, p):
    ep = hi.shape[0]
    (w1, b1), (w2, b2), (w3, b3) = p['mlp_phi']
    return pl.pallas_call(
        _t6p_body,
        grid=(ep // BE,),
        in_specs=[pl.BlockSpec((BE, F_H), lambda i: (i, 0)),
                  pl.BlockSpec((BE, F_H), lambda i: (i, 0)),
                  pl.BlockSpec((BE, F_H), lambda i: (i, 0)),
                  pl.BlockSpec((BE, 8), lambda i: (i, 0)),
                  pl.BlockSpec((F_H, 64), lambda i: (0, 0)),
                  pl.BlockSpec((F_H, 64), lambda i: (0, 0)),
                  pl.BlockSpec((F_H, 64), lambda i: (0, 0)),
                  pl.BlockSpec((8, 64), lambda i: (0, 0)),
                  pl.BlockSpec((1, 64), lambda i: (0, 0)),
                  pl.BlockSpec((64, 64), lambda i: (0, 0)),
                  pl.BlockSpec((1, 64), lambda i: (0, 0)),
                  pl.BlockSpec((64, 8), lambda i: (0, 0)),
                  pl.BlockSpec((1, 8), lambda i: (0, 0))],
        out_specs=pl.BlockSpec((BE, 16), lambda i: (i, 0)),
        out_shape=jax.ShapeDtypeStruct((ep, 16), jnp.float32),
    )(hi, hj, hk, trig, w1[0:64], w1[64:128], w1[128:192],
      jnp.pad(w1[192:194], ((0, 6), (0, 0))),
      b1[None, :], w2, b2[None, :], w3, b3[None, :])


def _t6c_body(si, sj, sk, sl, trig, wp1, wp2, wp3, wp4, b1cs,
              cw2, cb2, cw3, cb3, sw2, sb2, sw3, sb3, out):
    vi, vj, vk, vl = si[...], sj[...], sk[...], sl[...]
    s = jnp.concatenate([vi, vl], axis=0)
    t = jnp.concatenate([vj, vk], axis=0)
    u = jnp.concatenate([vk, vj], axis=0)
    v = jnp.concatenate([vl, vi], axis=0)
    ll = (_dot3(s, wp1[...]) + _dot3(t, wp2[...]) + _dot3(u, wp3[...])
          + _dot3(v, wp4[...]) + b1cs[...])
    zc = _mlp23(ll[:, 0:64], cw2[...], cb2[...], cw3[...], cb3[...])
    zs = _mlp23(ll[:, 64:128], sw2[...], sb2[...], sw3[...], sb3[...])
    c_t = zc[0:BE, 0:1] + zc[BE:2 * BE, 0:1]
    ps = zs[0:BE, 0:2] + zs[BE:2 * BE, 0:2]
    psn = jnp.sqrt(jnp.sum(ps * ps, axis=1, keepdims=True))
    psz = ps / jnp.maximum(psn, 1e-12)
    pc = psz[:, 0:1]
    psn_s = psz[:, 1:2]
    nc = 1.0 / (1.0 + jnp.exp(-c_t))
    cpsi = trig[:, 0:1]
    spsi = trig[:, 1:2]
    st0 = (cpsi * pc - spsi * psn_s) * nc
    st1 = (spsi * pc + cpsi * psn_s) * nc
    out[...] = jnp.concatenate(
        [c_t, psn, pc, psn_s, st0, st1, jnp.zeros((BE, 10), jnp.float32)],
        axis=1)


def _t6_psi(si, sj, sk, sl, trig, p):
    ep = si.shape[0]
    (cw1, cb1), (cw2, cb2), (cw3, cb3) = p['mlp_c']
    (sw1, sb1), (sw2, sb2), (sw3, sb3) = p['mlp_shift']
    wp = [jnp.concatenate([cw1[i * 64:(i + 1) * 64], sw1[i * 64:(i + 1) * 64]],
                          axis=1) for i in range(4)]
    b1cs = jnp.concatenate([cb1, sb1])[None, :]
    cw3p = jnp.pad(cw3, ((0, 0), (0, 7)))
    cb3p = jnp.pad(cb3, (0, 7))[None, :]
    sw3p = jnp.pad(sw3, ((0, 0), (0, 6)))
    sb3p = jnp.pad(sb3, (0, 6))[None, :]
    specs = ([pl.BlockSpec((BE, F_H), lambda i: (i, 0)) for _ in range(4)]
             + [pl.BlockSpec((BE, 8), lambda i: (i, 0))]
             + [pl.BlockSpec((F_H, 128), lambda i: (0, 0)) for _ in range(4)]
             + [pl.BlockSpec((1, 128), lambda i: (0, 0)),
                pl.BlockSpec((64, 64), lambda i: (0, 0)),
                pl.BlockSpec((1, 64), lambda i: (0, 0)),
                pl.BlockSpec((64, 8), lambda i: (0, 0)),
                pl.BlockSpec((1, 8), lambda i: (0, 0)),
                pl.BlockSpec((64, 64), lambda i: (0, 0)),
                pl.BlockSpec((1, 64), lambda i: (0, 0)),
                pl.BlockSpec((64, 8), lambda i: (0, 0)),
                pl.BlockSpec((1, 8), lambda i: (0, 0))])
    return pl.pallas_call(
        _t6c_body,
        grid=(ep // BE,),
        in_specs=specs,
        out_specs=pl.BlockSpec((BE, 16), lambda i: (i, 0)),
        out_shape=jax.ShapeDtypeStruct((ep, 16), jnp.float32),
    )(si, sj, sk, sl, trig, wp[0], wp[1], wp[2], wp[3], b1cs,
      cw2, cb2[None, :], cw3p, cb3p, sw2, sb2[None, :], sw3p, sb3p)


def _t6a_body(hx, hy, pa, pb, wa, wb, wr, b1, w2, b2, w3, b3, z_out, pool_out):
    pooled = pa[0:N_LS, 4:6] + pb[0:N_LS, 4:6]
    pool_out[...] = pooled
    radii = jnp.sqrt(jnp.sum(pooled * pooled, axis=1, keepdims=True))
    vx, vy = hx[...], hy[...]
    s = jnp.concatenate([vx, vy], axis=0)
    t = jnp.concatenate([vy, vx], axis=0)
    ex = radii * wr[...] + b1[...]
    l1 = _dot3(s, wa[...]) + _dot3(t, wb[...]) + jnp.concatenate([ex, ex], 0)
    zz = _mlp23(l1, w2[...], b2[...], w3[...], b3[...])
    z = zz[0:N_LS] + zz[N_LS:2 * N_LS]
    z_out[...] = jnp.concatenate([z, jnp.zeros((N_LS, 8), jnp.float32)],
                                 axis=1)


def _t6_alpha(hx, hy, pool_a, pool_b, p):
    (w1, b1), (w2, b2), (w3, b3) = p['mlp_alpha']
    return pl.pallas_call(
        _t6a_body,
        out_shape=(jax.ShapeDtypeStruct((N_LS, 16), jnp.float32),
                   jax.ShapeDtypeStruct((N_LS, 2), jnp.float32)),
    )(hx, hy, pool_a, pool_b, w1[0:64], w1[64:128], w1[128:129], b1[None, :],
      w2, b2[None, :], w3, b3[None, :])


def _t7_body(da, db, pa, pb, aa, ab, z_out):
    zd = da[0:N_MOL, 0:8] + db[0:N_MOL, 0:8]
    zp = pa[0:N_MOL, 0:8] + pb[0:N_MOL, 0:8]
    za = aa[0:N_MOL, 0:8] + ab[0:N_MOL, 0:8]
    z_out[...] = jnp.concatenate([zd, zp, za], axis=1)


def _t7_z(d2, p2, a2):
    return pl.pallas_call(
        _t7_body,
        out_shape=jax.ShapeDtypeStruct((N_MOL, 24), jnp.float32),
    )(d2[0], d2[1], p2[0], p2[1], a2[0], a2[1])


def kernel(x, edge_index, edge_attr, distances, distance_indices, phis,
           phi_indices, psis, psi_indices, node_map, LS_map, alpha_indices,
           params):
    p = params
    e = edge_index.shape[1]
    ep = ((e + 4095) // 4096) * 4096
    e2 = e + N_NODES
    e2p = ((e2 + 4095) // 4096) * 4096
    np_pad = N_NODES + 16
    nm_pad = N_MOL + 16
    nl_pad = N_LS + 16

    src = edge_index[0].astype(jnp.int32)
    dst = edge_index[1].astype(jnp.int32)
    ar = jnp.arange(N_NODES, dtype=jnp.int32)
    s2 = jnp.concatenate([src, ar])
    d2 = jnp.concatenate([dst, ar])

    xs = _gather_rows(x, _pad_idx(src, ep, 0))
    ea_p = _pad_rows(edge_attr, ep)
    msg = _t1_msg(ea_p, xs, p)
    agg = _scatter_add(msg, _pad_idx(dst, ep, N_NODES), np_pad)
    hh, packed = _t2_node(x, agg[0], agg[1], p)

    s2g = _pad_idx(s2, e2p, 0)
    d2g = _pad_idx(d2, e2p, 0)
    d2s = _pad_idx(d2, e2p, N_NODES)
    gs = _gather_rows(packed, s2g)
    gd = _gather_rows(packed, d2g)
    w = _t3_gatw(gs, gd)
    den = _scatter_add(w, d2s, np_pad)
    da = _gather_rows(den[0], d2g)
    db = _gather_rows(den[1], d2g)
    hhg = _gather_rows(hh, s2g)
    y = _t4_y(w, da, db, hhg)
    pp = _scatter_add(y, d2s, np_pad)
    h_nodes = _t5_h(pp[0], pp[1], p)

    di = distance_indices[0].astype(jnp.int32)
    dj = distance_indices[1].astype(jnp.int32)
    pi = phi_indices[0].astype(jnp.int32)
    pj = phi_indices[1].astype(jnp.int32)
    pk = phi_indices[2].astype(jnp.int32)
    si = psi_indices[0].astype(jnp.int32)
    sj = psi_indices[1].astype(jnp.int32)
    sk = psi_indices[2].astype(jnp.int32)
    sl = psi_indices[3].astype(jnp.int32)
    ax = alpha_indices[0].astype(jnp.int32)
    ay = alpha_indices[1].astype(jnp.int32)

    big_idx = jnp.concatenate([
        _pad_idx(di, ep, 0), _pad_idx(dj, ep, 0),
        _pad_idx(pi, ep, 0), _pad_idx(pj, ep, 0), _pad_idx(pk, ep, 0),
        _pad_idx(si, ep, 0), _pad_idx(sj, ep, 0), _pad_idx(sk, ep, 0),
        _pad_idx(sl, ep, 0), ax, ay])
    hg = _gather_rows(h_nodes, big_idx)
    hdi, hdj = hg[0:ep], hg[ep:2 * ep]
    hpi, hpj, hpk = hg[2 * ep:3 * ep], hg[3 * ep:4 * ep], hg[4 * ep:5 * ep]
    hsi, hsj = hg[5 * ep:6 * ep], hg[6 * ep:7 * ep]
    hsk, hsl = hg[7 * ep:8 * ep], hg[8 * ep:9 * ep]
    hax, hay = hg[9 * ep:9 * ep + N_LS], hg[9 * ep + N_LS:]

    d_col = _pad_rows(distances[:, None], ep)
    zero6 = jnp.zeros((phis.shape[0], 6), jnp.float32)
    trig_phi = _pad_rows(
        jnp.concatenate([jnp.cos(phis)[:, None], jnp.sin(phis)[:, None],
                         zero6], 1), ep)
    trig_psi = _pad_rows(
        jnp.concatenate([jnp.cos(psis)[:, None], jnp.sin(psis)[:, None],
                         zero6], 1), ep)

    z_d16 = _t6_d(hdi, hdj, d_col, p)
    z_p16 = _t6_phi(hpi, hpj, hpk, trig_phi, p)
    psi16 = _t6_psi(hsi, hsj, hsk, hsl, trig_psi, p)

    pooled2 = _scatter_add(psi16, _pad_idx(LS_map.astype(jnp.int32), ep, N_LS),
                           nl_pad)
    z_a16, pooled = _t6_alpha(hax, hay, pooled2[0], pooled2[1], p)

    map_ext = jnp.pad(node_map.astype(jnp.int32), (0, 16),
                      constant_values=N_MOL)
    mol_d = _gather_rows(map_ext, _pad_idx(di, ep, N_NODES))
    mol_p = _gather_rows(map_ext, _pad_idx(pi, ep, N_NODES))
    mol_a = _gather_rows(map_ext, ax)
    zd2 = _scatter_add(z_d16, mol_d, nm_pad)
    zp2 = _scatter_add(z_p16, mol_p, nm_pad)
    za2 = _scatter_add(z_a16, mol_a, nm_pad)
    z = _t7_z(zd2, zp2, za2)

    c_tensor = psi16[0:e, 0:1]
    psn = psi16[0:e, 1:2]
    phase_cos = psi16[0:e, 2]
    phase_sin = psi16[0:e, 3]
    cs_psi = trig_psi[0:e, 0:2]
    z_alpha = z_a16[:, 0:8]
    return (z, psn, z_alpha, c_tensor, phase_cos, phase_sin, cs_psi, pooled)
